# baseline (speedup 1.0000x reference)
import functools

import jax
import jax.numpy as jnp
from jax import lax
from jax.experimental import pallas as pl
from jax.experimental.pallas import tpu as pltpu
from jax.experimental.pallas import tpu_sc as plsc

_NH = 16
_HH = 8
_NRD = (2 * 8 - 1) ** 3 + 3
_VP1 = 8 * 8 * 8 + 1
_BR = 8
_NBLK = (_VP1 - 1) // _BR
_NW = 32

_mesh = plsc.VectorSubcoreMesh(core_axis_name="c", subcore_axis_name="s")


@functools.partial(
    pl.kernel,
    mesh=_mesh,
    out_type=jax.ShapeDtypeStruct((_VP1, _NH, _VP1), jnp.float32),
    compiler_params=pltpu.CompilerParams(needs_layout_passes=False),
    scratch_types=[
        pltpu.VMEM((_HH * _NRD,), jnp.float32),
        pltpu.VMEM((_BR, _HH, _VP1), jnp.float32),
        pltpu.VMEM((_BR, _HH, _VP1), jnp.float32),
        pltpu.VMEM((512,), jnp.int32),
        pltpu.SemaphoreType.DMA,
        pltpu.SemaphoreType.DMA,
    ],
)
def _sc_gather(tab_hbm, out_hbm, tab_v, buf0, buf1, bsum_v, sem0, sem1):
    w = lax.axis_index("s") * 2 + lax.axis_index("c")
    h0 = (w & 1) * _HH
    pltpu.sync_copy(tab_hbm.at[pl.ds(h0 * _NRD, _HH * _NRD)], tab_v)

    lane = lax.broadcasted_iota(jnp.int32, (16,), 0)
    lane0 = lane < 1
    col_last = jnp.full((16,), _VP1 - 1, jnp.int32)

    def gather_vec(vec):
        return [plsc.load_gather(tab_v, [vec + h * _NRD]) for h in range(_HH)]

    def row_consts(i):
        iv = jnp.broadcast_to(jnp.int32(0) + i, (16,))
        a = iv - 1
        s = (((a >> 6) + 7) * 225 + (((a >> 3) & 7) + 7) * 15
             + ((a & 7) + 7))
        return s, iv == 0

    for v in range(32):
        b = (v * 16 - 1) + lane
        bsum_v[pl.ds(v * 16, 16)] = ((b >> 6) * 225 + ((b >> 3) & 7) * 15
                                     + (b & 7))
    col0 = lane < 1

    def fill(buf, row_of_dyn, row_of_vec, nrows, r0=0):
        def rbody(rr, carry):
            s, is_row0 = row_consts(row_of_dyn(rr))
            row0_v0 = jnp.where(col0, _NRD - 1, _NRD - 3)
            for v in range(32):
                idx = s - bsum_v[pl.ds(v * 16, 16)]
                if v == 0:
                    idx = jnp.where(col0, _NRD - 2, idx)
                    idx = jnp.where(is_row0, row0_v0, idx)
                else:
                    idx = jnp.where(is_row0, _NRD - 3, idx)
                vals = gather_vec(idx)
                for h in range(_HH):
                    buf[r0 + rr, h, pl.ds(v * 16, 16)] = vals[h]
            return carry
        lax.fori_loop(0, nrows, rbody, 0)

        rvec = lane & (_BR - 1)
        iv = row_of_vec(rvec)
        a = iv - 1
        s = (((a >> 6) + 7) * 225 + (((a >> 3) & 7) + 7) * 15 + ((a & 7) + 7))
        bsum_t = 7 * 225 + 7 * 15 + 7
        idx = jnp.where(iv == 0, _NRD - 3, s - bsum_t)
        rowm = jnp.minimum(rvec, nrows - 1) + r0
        maskr = rvec < nrows
        for h in range(_HH):
            vals = plsc.load_gather(tab_v, [idx + h * _NRD])
            plsc.store_scatter(buf, [rowm, jnp.full((16,), h, jnp.int32),
                                     col_last], vals, mask=maskr)

    bufs, sems, pending = (buf0, buf1), (sem0, sem1), [None, None]
    for k in range(4):
        p = k & 1
        if pending[p] is not None:
            pending[p].wait()
        blk = (w >> 1) * 4 + k
        fill(bufs[p], lambda rr, blk=blk: blk * _BR + rr,
             lambda rv, blk=blk: blk * _BR + rv, _BR)
        pending[p] = pltpu.async_copy(
            bufs[p],
            out_hbm.at[pl.ds(blk * _BR, _BR), pl.ds(h0, _HH), :],
            sems[p])
    for p in (0, 1):
        pending[p].wait()

    @pl.when(w < 2)
    def _last_row():
        fill(buf0, lambda rr: _NBLK * _BR,
             lambda rv: jnp.full((16,), _NBLK * _BR, jnp.int32), 1)
        pltpu.sync_copy(
            buf0.at[pl.ds(0, 1), :, :],
            out_hbm.at[pl.ds(_NBLK * _BR, 1), pl.ds(h0, _HH), :])


def kernel(inputs, rpbt):
    del inputs
    tab = jnp.transpose(rpbt).reshape(-1)
    out = _sc_gather(tab)
    return jnp.transpose(out, (1, 0, 2))


# speedup vs baseline: 1.2753x; 1.2753x over previous
import functools

import jax
import jax.numpy as jnp
from jax import lax
from jax.experimental import pallas as pl
from jax.experimental.pallas import tpu as pltpu
from jax.experimental.pallas import tpu_sc as plsc

_NH = 16
_HH = 8
_NRD = (2 * 8 - 1) ** 3 + 3
_VP1 = 8 * 8 * 8 + 1
_BR = 8
_NBLK = (_VP1 - 1) // _BR
_NW = 32

_mesh = plsc.VectorSubcoreMesh(core_axis_name="c", subcore_axis_name="s")


@functools.partial(
    pl.kernel,
    mesh=_mesh,
    out_type=jax.ShapeDtypeStruct((_VP1, _NH, _VP1), jnp.float32),
    compiler_params=pltpu.CompilerParams(needs_layout_passes=False),
    scratch_types=[
        pltpu.VMEM((_HH * _NRD,), jnp.float32),
        pltpu.VMEM((_BR, _HH, _VP1), jnp.float32),
        pltpu.VMEM((_BR, _HH, _VP1), jnp.float32),
        pltpu.SemaphoreType.DMA,
        pltpu.SemaphoreType.DMA,
    ],
)
def _sc_gather(tab_hbm, out_hbm, tab_v, buf0, buf1, sem0, sem1):
    w = lax.axis_index("s") * 2 + lax.axis_index("c")
    h0 = (w & 1) * _HH
    pltpu.sync_copy(tab_hbm.at[pl.ds(h0 * _NRD, _HH * _NRD)], tab_v)

    lane = lax.broadcasted_iota(jnp.int32, (16,), 0)
    lane0 = lane < 1
    col_last = jnp.full((16,), _VP1 - 1, jnp.int32)

    def gather_vec(vec):
        return [plsc.load_gather(tab_v, [vec + h * _NRD]) for h in range(_HH)]

    def row_consts(i):
        iv = jnp.broadcast_to(jnp.int32(0) + i, (16,))
        a = iv - 1
        s = (((a >> 6) + 7) * 225 + (((a >> 3) & 7) + 7) * 15
             + ((a & 7) + 7))
        return s, iv == 0

    def fill(buf, row_of, row_of_vec, nrows, r0=0):
        consts = [row_consts(row_of(r)) for r in range(nrows)]

        def _body(v, carry):
            for u in range(2):
                j = (v * 2 + u) * 16 + lane
                b = j - 1
                bsum = (b >> 6) * 225 + ((b >> 3) & 7) * 15 + (b & 7)
                col0 = j == 0
                row0_val = jnp.where(col0, _NRD - 1, _NRD - 3)
                for r in range(nrows):
                    s, is_row0 = consts[r]
                    idx = jnp.where(col0, _NRD - 2, s - bsum)
                    idx = jnp.where(is_row0, row0_val, idx)
                    vals = gather_vec(idx)
                    for h in range(_HH):
                        buf[r0 + r, h, pl.ds((v * 2 + u) * 16, 16)] = vals[h]
            return carry
        lax.fori_loop(0, (_VP1 - 1) // 32, _body, 0)

        rvec = lane & (_BR - 1)
        iv = row_of_vec(rvec)
        a = iv - 1
        s = (((a >> 6) + 7) * 225 + (((a >> 3) & 7) + 7) * 15 + ((a & 7) + 7))
        bsum_t = 7 * 225 + 7 * 15 + 7
        idx = jnp.where(iv == 0, _NRD - 3, s - bsum_t)
        rowm = jnp.minimum(rvec, nrows - 1) + r0
        maskr = rvec < nrows
        for h in range(_HH):
            vals = plsc.load_gather(tab_v, [idx + h * _NRD])
            plsc.store_scatter(buf, [rowm, jnp.full((16,), h, jnp.int32),
                                     col_last], vals, mask=maskr)

    bufs, sems, pending = (buf0, buf1), (sem0, sem1), [None, None]
    for k in range(4):
        p = k & 1
        if pending[p] is not None:
            pending[p].wait()
        blk = (w >> 1) * 4 + k
        fill(bufs[p], lambda r, blk=blk: blk * _BR + r,
             lambda rv, blk=blk: blk * _BR + rv, _BR)
        pending[p] = pltpu.async_copy(
            bufs[p],
            out_hbm.at[pl.ds(blk * _BR, _BR), pl.ds(h0, _HH), :],
            sems[p])
    for p in (0, 1):
        pending[p].wait()

    @pl.when(w < 2)
    def _last_row():
        fill(buf0, lambda r: _NBLK * _BR,
             lambda rv: jnp.full((16,), _NBLK * _BR, jnp.int32), 1)
        pltpu.sync_copy(
            buf0.at[pl.ds(0, 1), :, :],
            out_hbm.at[pl.ds(_NBLK * _BR, 1), pl.ds(h0, _HH), :])


def kernel(inputs, rpbt):
    del inputs
    tab = jnp.transpose(rpbt).reshape(-1)
    out = _sc_gather(tab)
    return jnp.transpose(out, (1, 0, 2))
